# baseline (speedup 1.0000x reference)
import functools

import jax
import jax.numpy as jnp
from jax import lax
from jax.experimental import pallas as pl
from jax.experimental.pallas import tpu as pltpu
from jax.experimental.pallas import tpu_sc as plsc

NC = 2
NS = 16
NW = NC * NS

CH = 128


@functools.partial(jax.jit, static_argnames=("n_per_w", "n_ch", "d"))
def _sc_gather(idx3, table, n_per_w, n_ch, d):
    n = idx3.shape[0] * idx3.shape[1] * idx3.shape[2]

    mesh = plsc.VectorSubcoreMesh(
        core_axis_name="c", subcore_axis_name="s", num_cores=NC, num_subcores=NS
    )

    @functools.partial(
        pl.kernel,
        mesh=mesh,
        out_type=jax.ShapeDtypeStruct((n, d), jnp.float32),
        scratch_types=[
            pltpu.VMEM((n_ch, CH), jnp.int32),
            pltpu.VMEM((2, CH, d), jnp.float32),
            pltpu.SemaphoreType.DMA,
            pltpu.SemaphoreType.DMA,
        ],
        compiler_params=pltpu.CompilerParams(use_tc_tiling_on_sc=False),
    )
    def k(idx_hbm, table_hbm, out_hbm, idx_v, rows_v, gsem, wsem):
        wid = lax.axis_index("s") * NC + lax.axis_index("c")
        base = wid * n_per_w
        pltpu.sync_copy(idx_hbm.at[wid], idx_v)

        def start_gather(j, b):
            return pltpu.async_copy(table_hbm.at[idx_v.at[j]], rows_v.at[b], gsem)

        def start_write(j, b):
            return pltpu.async_copy(
                rows_v.at[b], out_hbm.at[pl.ds(base + j * CH, CH)], wsem
            )

        start_gather(0, 0)

        def body(j, _):
            b = lax.rem(j, 2)
            start_gather(j + 1, 1 - b)
            pltpu.make_async_copy(
                table_hbm.at[idx_v.at[j]], rows_v.at[b], gsem
            ).wait()

            @pl.when(j > 0)
            def _():
                pltpu.make_async_copy(
                    rows_v.at[1 - b],
                    out_hbm.at[pl.ds(base + (j - 1) * CH, CH)],
                    wsem,
                ).wait()

            start_write(j, b)
            return ()

        lax.fori_loop(0, n_ch - 1, body, (), unroll=False)

        last = n_ch - 1
        b = lax.rem(last, 2)
        pltpu.make_async_copy(table_hbm.at[idx_v.at[last]], rows_v.at[b], gsem).wait()
        pltpu.make_async_copy(
            rows_v.at[1 - b], out_hbm.at[pl.ds(base + (last - 1) * CH, CH)], wsem
        ).wait()
        pltpu.sync_copy(rows_v.at[b], out_hbm.at[pl.ds(base + last * CH, CH)])

    return k(idx3, table)


def kernel(input_ids, embedding):
    batch, hist = input_ids.shape
    vocab, d = embedding.shape
    n = batch * hist
    assert n % (NW * CH) == 0
    n_per_w = n // NW
    n_ch = n_per_w // CH
    idx3 = input_ids.reshape(NW, n_ch, CH)
    out = _sc_gather(idx3, embedding, n_per_w, n_ch, d)
    return out.reshape(batch, hist, d)


# speedup vs baseline: 1.0053x; 1.0053x over previous
import functools

import jax
import jax.numpy as jnp
from jax import lax
from jax.experimental import pallas as pl
from jax.experimental.pallas import tpu as pltpu
from jax.experimental.pallas import tpu_sc as plsc

NC = 2
NS = 16
NW = NC * NS

CH = 128
KG = 4
SUP = CH * KG
NBUF = 2


@functools.partial(jax.jit, static_argnames=("n_per_w", "n_ch", "d"))
def _sc_gather(idx3, table, n_per_w, n_ch, d):
    n = idx3.shape[0] * idx3.shape[1] * idx3.shape[2]
    n_sup = n_per_w // SUP

    mesh = plsc.VectorSubcoreMesh(
        core_axis_name="c", subcore_axis_name="s", num_cores=NC, num_subcores=NS
    )

    @functools.partial(
        pl.kernel,
        mesh=mesh,
        out_type=jax.ShapeDtypeStruct((n, d), jnp.float32),
        scratch_types=[
            pltpu.VMEM((n_ch, CH), jnp.int32),
            pltpu.VMEM((NBUF, SUP, d), jnp.float32),
            pltpu.SemaphoreType.DMA,
            pltpu.SemaphoreType.DMA,
        ],
        compiler_params=pltpu.CompilerParams(use_tc_tiling_on_sc=False),
    )
    def k(idx_hbm, table_hbm, out_hbm, idx_v, rows_v, gsem, wsem):
        wid = lax.axis_index("s") * NC + lax.axis_index("c")
        base = wid * n_per_w
        pltpu.sync_copy(idx_hbm.at[wid], idx_v)

        def fire_gathers(t, b):
            for g in range(KG):
                pltpu.async_copy(
                    table_hbm.at[idx_v.at[t * KG + g]],
                    rows_v.at[b].at[pl.ds(g * CH, CH)],
                    gsem,
                )

        def drain_gathers(t, b):
            for g in range(KG):
                pltpu.make_async_copy(
                    table_hbm.at[idx_v.at[t * KG + g]],
                    rows_v.at[b].at[pl.ds(g * CH, CH)],
                    gsem,
                ).wait()

        def write_desc(t, b):
            return pltpu.make_async_copy(
                rows_v.at[b], out_hbm.at[pl.ds(base + t * SUP, SUP)], wsem
            )

        @pl.loop(0, n_sup, step=NBUF)
        def _(t0):
            for b in range(NBUF):
                t = t0 + b

                @pl.when(t >= NBUF)
                def _():
                    write_desc(t - NBUF, b).wait()

                fire_gathers(t, b)
                drain_gathers(t, b)
                write_desc(t, b).start()

        for b in range(NBUF):
            write_desc(n_sup - NBUF + b, b).wait()

    return k(idx3, table)


def kernel(input_ids, embedding):
    batch, hist = input_ids.shape
    vocab, d = embedding.shape
    n = batch * hist
    assert n % (NW * SUP * NBUF) == 0
    n_per_w = n // NW
    n_ch = n_per_w // CH
    idx3 = input_ids.reshape(NW, n_ch, CH)
    out = _sc_gather(idx3, embedding, n_per_w, n_ch, d)
    return out.reshape(batch, hist, d)
